# baseline (speedup 1.0000x reference)
import functools

import jax
import jax.numpy as jnp
from jax import lax
from jax.experimental import pallas as pl
from jax.experimental.pallas import tpu as pltpu
from jax.experimental.pallas import tpu_sc as plsc

_B, _S, _E = 4096, 200, 64
_NC, _NS = 2, 16
_NW = _NC * _NS
_BPW = _B // _NW
_W0 = 128
_W1 = _S - _W0


_V = 1000000
_VB = 1024
_NBLK = (_V + _VB - 1) // _VB


def _untile_body(in_ref, out_ref):
    x = in_ref[...]
    out_ref[...] = (x.reshape(_E, _VB // 2, 2).transpose(1, 2, 0)
                    .reshape(_VB // 2, 2 * _E))


def _untile(tbl_t):
    return pl.pallas_call(
        _untile_body,
        grid=(_NBLK,),
        in_specs=[pl.BlockSpec((_E, _VB), lambda i: (0, i))],
        out_specs=pl.BlockSpec((_VB // 2, 2 * _E), lambda i: (i, 0)),
        out_shape=jax.ShapeDtypeStruct((_V // 2, 2 * _E), jnp.float32),
    )(tbl_t)


def _bag_sums(emb_table, idx_flat):
    mesh = plsc.VectorSubcoreMesh(
        core_axis_name="c", subcore_axis_name="s",
        num_cores=_NC, num_subcores=_NS)

    @functools.partial(
        pl.kernel,
        out_type=jax.ShapeDtypeStruct((_B, _E), jnp.float32),
        mesh=mesh,
        compiler_params=pltpu.CompilerParams(use_tc_tiling_on_sc=False),
        scratch_types=[
            pltpu.VMEM((_BPW * _S,), jnp.int32),
            pltpu.VMEM((_S, _E), jnp.float32),
            pltpu.VMEM((_BPW, _E), jnp.float32),
            pltpu.SemaphoreType.DMA,
        ],
    )
    def k(table_hbm, idx_hbm, out_hbm, idx_v, rows_v, out_v, sem):
        wid = lax.axis_index("s") * _NC + lax.axis_index("c")
        base = wid * _BPW
        pltpu.sync_copy(idx_hbm.at[pl.ds(base * _S, _BPW * _S)], idx_v)

        @pl.loop(0, _BPW)
        def _row(r):
            off = r * _S
            c1 = pltpu.async_copy(
                table_hbm.at[idx_v.at[pl.ds(off, _W0)]],
                rows_v.at[pl.ds(0, _W0)], sem)
            c2 = pltpu.async_copy(
                table_hbm.at[idx_v.at[pl.ds(off + _W0, _W1)]],
                rows_v.at[pl.ds(_W0, _W1)], sem)
            c1.wait()
            c2.wait()

            zero = jnp.zeros((16,), jnp.float32)

            def acc_body(i, carry):
                cs = list(carry)
                for j in range(4):
                    row = i * 4 + j
                    for c in range(4):
                        cs[c] = cs[c] + rows_v[row, pl.ds(c * 16, 16)]
                return tuple(cs)

            acc = lax.fori_loop(0, _S // 4, acc_body, (zero, zero, zero, zero))
            for c in range(4):
                out_v[r, pl.ds(c * 16, 16)] = acc[c]

        pltpu.sync_copy(out_v, out_hbm.at[pl.ds(base, _BPW)])

    return k(emb_table, idx_flat)


def _tail_body(seq_ref, sums_ref, cat_ref, Wc_ref, bc_ref, gc_ref, bec_ref,
               W1_ref, b1_ref, g1_ref, be1_ref, W2_ref, b2_ref, g2_ref,
               be2_ref, W3_ref, b3_ref, out_ref):
    def mm(x, w_ref_val):
        return lax.dot_general(x, w_ref_val, (((1,), (1,)), ((), ())),
                               preferred_element_type=jnp.float32)

    def bn(x, g, b):
        m = jnp.mean(x, axis=0, keepdims=True)
        v = jnp.mean((x - m) ** 2, axis=0, keepdims=True)
        return g * (x - m) * lax.rsqrt(v + 1e-5) + b

    seq = seq_ref[...]
    mask = (seq != 0).astype(jnp.float32)
    cnt = lax.dot_general(mask, jnp.ones((_S, _E), jnp.float32),
                          (((1,), (0,)), ((), ())),
                          preferred_element_type=jnp.float32)
    text_feat = sums_ref[...] / (cnt + 1e-9)

    cat = mm(cat_ref[...], Wc_ref[...]) + bc_ref[...]
    cat = jnp.maximum(bn(cat, gc_ref[...], bec_ref[...]), 0.0)

    h = (mm(text_feat, W1_ref[:, : _E]) + mm(cat, W1_ref[:, _E:])
         + b1_ref[...])
    h = jnp.maximum(bn(h, g1_ref[...], be1_ref[...]), 0.0)
    h = mm(h, W2_ref[...]) + b2_ref[...]
    h = jnp.maximum(bn(h, g2_ref[...], be2_ref[...]), 0.0)
    out_ref[...] = (jnp.sum(h * W3_ref[...], axis=1, keepdims=True)
                    + b3_ref[0, 0])


def _tail(seq_i32, sums, cat_features, W_cat, b_cat, g_cat, be_cat,
          W1, b1, g1, be1, W2, b2, g2, be2, W3, b3, interpret=False):
    row = lambda x: x.reshape(1, -1)
    return pl.pallas_call(
        _tail_body,
        out_shape=jax.ShapeDtypeStruct((_B, 1), jnp.float32),
        interpret=interpret,
    )(seq_i32, sums, cat_features, W_cat, row(b_cat), row(g_cat),
      row(be_cat), W1, row(b1), row(g1), row(be1), W2, row(b2), row(g2),
      row(be2), W3, row(b3))


def kernel(text_seq, cat_features, emb_table, W_cat, b_cat, g_cat, be_cat,
           W1, b1, g1, be1, W2, b2, g2, be2, W3, b3):
    seq_i32 = text_seq.astype(jnp.int32)
    tbl_lin = _untile(emb_table.T).reshape(_V, _E)
    sums = _bag_sums(tbl_lin, seq_i32.reshape(_B * _S))
    return _tail(seq_i32, sums, cat_features, W_cat, b_cat, g_cat, be_cat,
                 W1, b1, g1, be1, W2, b2, g2, be2, W3, b3)


# speedup vs baseline: 17.3684x; 17.3684x over previous
import functools

import jax
import jax.numpy as jnp
from jax import lax
from jax.experimental import pallas as pl
from jax.experimental.pallas import tpu as pltpu
from jax.experimental.pallas import tpu_sc as plsc

_B, _S, _E = 4096, 200, 64
_NC, _NS = 2, 16
_NW = _NC * _NS
_BPW = _B // _NW
_W0 = 128
_W1 = _S - _W0


_V = 1000000
_H = 524288
_VB = 4096
_NBLK = _H // _VB


def _untile_body(in0_ref, in1_ref, out_ref):
    out_ref[...] = jnp.concatenate([in0_ref[...].T, in1_ref[...].T], axis=1)


def _untile(tbl_t):
    return pl.pallas_call(
        _untile_body,
        grid=(_NBLK,),
        in_specs=[pl.BlockSpec((_E, _VB), lambda i: (0, i)),
                  pl.BlockSpec((_E, _VB),
                               lambda i: (0, jnp.minimum(i + _NBLK,
                                                         _V // _VB - 1)))],
        out_specs=pl.BlockSpec((_VB, 2 * _E), lambda i: (i, 0)),
        out_shape=jax.ShapeDtypeStruct((_H, 2 * _E), jnp.float32),
    )(tbl_t, tbl_t)


def _bag_sums(emb_table, idx_flat):
    mesh = plsc.VectorSubcoreMesh(
        core_axis_name="c", subcore_axis_name="s",
        num_cores=_NC, num_subcores=_NS)

    @functools.partial(
        pl.kernel,
        out_type=jax.ShapeDtypeStruct((_B, _E), jnp.float32),
        mesh=mesh,
        compiler_params=pltpu.CompilerParams(use_tc_tiling_on_sc=False),
        scratch_types=[
            pltpu.VMEM((_BPW * _S,), jnp.int32),
            pltpu.VMEM((_S, _E), jnp.float32),
            pltpu.VMEM((_BPW, _E), jnp.float32),
            pltpu.SemaphoreType.DMA,
        ],
    )
    def k(table_hbm, idx_hbm, out_hbm, idx_v, rows_v, out_v, sem):
        wid = lax.axis_index("s") * _NC + lax.axis_index("c")
        base = wid * _BPW
        pltpu.sync_copy(idx_hbm.at[pl.ds(base * _S, _BPW * _S)], idx_v)

        @pl.loop(0, _BPW)
        def _row(r):
            off = r * _S
            c1 = pltpu.async_copy(
                table_hbm.at[idx_v.at[pl.ds(off, _W0)]],
                rows_v.at[pl.ds(0, _W0)], sem)
            c2 = pltpu.async_copy(
                table_hbm.at[idx_v.at[pl.ds(off + _W0, _W1)]],
                rows_v.at[pl.ds(_W0, _W1)], sem)
            c1.wait()
            c2.wait()

            zero = jnp.zeros((16,), jnp.float32)

            def acc_body(i, carry):
                cs = list(carry)
                for j in range(4):
                    row = i * 4 + j
                    for c in range(4):
                        cs[c] = cs[c] + rows_v[row, pl.ds(c * 16, 16)]
                return tuple(cs)

            acc = lax.fori_loop(0, _S // 4, acc_body, (zero, zero, zero, zero))
            for c in range(4):
                out_v[r, pl.ds(c * 16, 16)] = acc[c]

        pltpu.sync_copy(out_v, out_hbm.at[pl.ds(base, _BPW)])

    return k(emb_table, idx_flat)


def _tail_body(seq_ref, sums_ref, cat_ref, Wc_ref, bc_ref, gc_ref, bec_ref,
               W1_ref, b1_ref, g1_ref, be1_ref, W2_ref, b2_ref, g2_ref,
               be2_ref, W3_ref, b3_ref, out_ref):
    def mm(x, w_ref_val):
        return lax.dot_general(x, w_ref_val, (((1,), (1,)), ((), ())),
                               preferred_element_type=jnp.float32)

    def bn(x, g, b):
        m = jnp.mean(x, axis=0, keepdims=True)
        v = jnp.mean((x - m) ** 2, axis=0, keepdims=True)
        return g * (x - m) * lax.rsqrt(v + 1e-5) + b

    seq = seq_ref[...]
    mask = (seq != 0).astype(jnp.float32)
    cnt = lax.dot_general(mask, jnp.ones((_S, _E), jnp.float32),
                          (((1,), (0,)), ((), ())),
                          preferred_element_type=jnp.float32)
    text_feat = sums_ref[...] / (cnt + 1e-9)

    cat = mm(cat_ref[...], Wc_ref[...]) + bc_ref[...]
    cat = jnp.maximum(bn(cat, gc_ref[...], bec_ref[...]), 0.0)

    h = (mm(text_feat, W1_ref[:, : _E]) + mm(cat, W1_ref[:, _E:])
         + b1_ref[...])
    h = jnp.maximum(bn(h, g1_ref[...], be1_ref[...]), 0.0)
    h = mm(h, W2_ref[...]) + b2_ref[...]
    h = jnp.maximum(bn(h, g2_ref[...], be2_ref[...]), 0.0)
    out_ref[...] = (jnp.sum(h * W3_ref[...], axis=1, keepdims=True)
                    + b3_ref[0, 0])


def _tail(seq_i32, sums, cat_features, W_cat, b_cat, g_cat, be_cat,
          W1, b1, g1, be1, W2, b2, g2, be2, W3, b3, interpret=False):
    row = lambda x: x.reshape(1, -1)
    return pl.pallas_call(
        _tail_body,
        out_shape=jax.ShapeDtypeStruct((_B, 1), jnp.float32),
        interpret=interpret,
    )(seq_i32, sums, cat_features, W_cat, row(b_cat), row(g_cat),
      row(be_cat), W1, row(b1), row(g1), row(be1), W2, row(b2), row(g2),
      row(be2), W3, row(b3))


def kernel(text_seq, cat_features, emb_table, W_cat, b_cat, g_cat, be_cat,
           W1, b1, g1, be1, W2, b2, g2, be2, W3, b3):
    seq_i32 = text_seq.astype(jnp.int32)
    tbl_lin = _untile(emb_table.T).reshape(2 * _H, _E)
    gidx = jnp.where(seq_i32 < _H, 2 * seq_i32, 2 * seq_i32 - (2 * _H - 1))
    sums = _bag_sums(tbl_lin, gidx.reshape(_B * _S))
    return _tail(seq_i32, sums, cat_features, W_cat, b_cat, g_cat, be_cat,
                 W1, b1, g1, be1, W2, b2, g2, be2, W3, b3)


# speedup vs baseline: 24.8640x; 1.4316x over previous
import functools

import jax
import jax.numpy as jnp
from jax import lax
from jax.experimental import pallas as pl
from jax.experimental.pallas import tpu as pltpu
from jax.experimental.pallas import tpu_sc as plsc

_B, _S, _E = 4096, 200, 64
_NC, _NS = 2, 16
_NW = _NC * _NS
_BPW = _B // _NW
_W0 = 128
_W1 = _S - _W0


_V = 1000000
_H = 524288
_VB = 8192
_NBLK = _H // _VB


def _untile_body(in0_ref, in1_ref, out_ref):
    out_ref[...] = jnp.concatenate([in0_ref[...].T, in1_ref[...].T], axis=1)


def _untile(tbl_t):
    return pl.pallas_call(
        _untile_body,
        grid=(_NBLK,),
        in_specs=[pl.BlockSpec((_E, _VB), lambda i: (0, i)),
                  pl.BlockSpec((_E, _VB),
                               lambda i: (0, jnp.minimum(i + _NBLK,
                                                         _V // _VB - 1)))],
        out_specs=pl.BlockSpec((_VB, 2 * _E), lambda i: (i, 0)),
        out_shape=jax.ShapeDtypeStruct((_H, 2 * _E), jnp.float32),
    )(tbl_t, tbl_t)


def _bag_sums(emb_table, idx_flat):
    mesh = plsc.VectorSubcoreMesh(
        core_axis_name="c", subcore_axis_name="s",
        num_cores=_NC, num_subcores=_NS)

    @functools.partial(
        pl.kernel,
        out_type=jax.ShapeDtypeStruct((_B, _E), jnp.float32),
        mesh=mesh,
        compiler_params=pltpu.CompilerParams(use_tc_tiling_on_sc=False),
        scratch_types=[
            pltpu.VMEM((_BPW * _S,), jnp.int32),
            pltpu.VMEM((2, _S, _E), jnp.float32),
            pltpu.VMEM((_BPW, _E), jnp.float32),
            pltpu.SemaphoreType.DMA,
            pltpu.SemaphoreType.DMA,
        ],
    )
    def k(table_hbm, idx_hbm, out_hbm, idx_v, rows_v, out_v, sem0, sem1):
        wid = lax.axis_index("s") * _NC + lax.axis_index("c")
        base = wid * _BPW
        pltpu.sync_copy(idx_hbm.at[pl.ds(base * _S, _BPW * _S)], idx_v)

        def descs(r, buf, sem):
            off = r * _S
            return (
                pltpu.make_async_copy(
                    table_hbm.at[idx_v.at[pl.ds(off, _W0)]],
                    rows_v.at[buf].at[pl.ds(0, _W0)], sem),
                pltpu.make_async_copy(
                    table_hbm.at[idx_v.at[pl.ds(off + _W0, _W1)]],
                    rows_v.at[buf].at[pl.ds(_W0, _W1)], sem),
            )

        def issue(r, buf, sem):
            for d in descs(r, buf, sem):
                d.start()

        def wait(r, buf, sem):
            for d in descs(r, buf, sem):
                d.wait()

        def accum(r, buf):
            zero = jnp.zeros((16,), jnp.float32)

            def acc_body(i, carry):
                cs = list(carry)
                for j in range(4):
                    row = i * 4 + j
                    for c in range(4):
                        cs[c] = cs[c] + rows_v[buf, row, pl.ds(c * 16, 16)]
                return tuple(cs)

            acc = lax.fori_loop(0, _S // 4, acc_body, (zero,) * 4)
            for c in range(4):
                out_v[r, pl.ds(c * 16, 16)] = acc[c]

        issue(0, 0, sem0)
        issue(1, 1, sem1)

        @pl.loop(0, _BPW // 2)
        def _pair(i):
            r0 = 2 * i
            wait(r0, 0, sem0)

            @pl.when(i < _BPW // 2 - 1)
            def _():
                issue(r0 + 2, 0, sem0)

            accum(r0, 0)
            r1 = r0 + 1
            wait(r1, 1, sem1)

            @pl.when(i < _BPW // 2 - 1)
            def _():
                issue(r1 + 2, 1, sem1)

            accum(r1, 1)

        pltpu.sync_copy(out_v, out_hbm.at[pl.ds(base, _BPW)])

    return k(emb_table, idx_flat)


def _tail_body(seq_ref, sums_ref, cat_ref, Wc_ref, bc_ref, gc_ref, bec_ref,
               W1_ref, b1_ref, g1_ref, be1_ref, W2_ref, b2_ref, g2_ref,
               be2_ref, W3_ref, b3_ref, out_ref):
    def mm(x, w_ref_val):
        return lax.dot_general(x, w_ref_val, (((1,), (1,)), ((), ())),
                               preferred_element_type=jnp.float32)

    def bn(x, g, b):
        m = jnp.mean(x, axis=0, keepdims=True)
        v = jnp.mean((x - m) ** 2, axis=0, keepdims=True)
        return g * (x - m) * lax.rsqrt(v + 1e-5) + b

    seq = seq_ref[...]
    mask = (seq != 0).astype(jnp.float32)
    cnt = lax.dot_general(mask, jnp.ones((_S, _E), jnp.float32),
                          (((1,), (0,)), ((), ())),
                          preferred_element_type=jnp.float32)
    text_feat = sums_ref[...] / (cnt + 1e-9)

    cat = mm(cat_ref[...], Wc_ref[...]) + bc_ref[...]
    cat = jnp.maximum(bn(cat, gc_ref[...], bec_ref[...]), 0.0)

    h = (mm(text_feat, W1_ref[:, : _E]) + mm(cat, W1_ref[:, _E:])
         + b1_ref[...])
    h = jnp.maximum(bn(h, g1_ref[...], be1_ref[...]), 0.0)
    h = mm(h, W2_ref[...]) + b2_ref[...]
    h = jnp.maximum(bn(h, g2_ref[...], be2_ref[...]), 0.0)
    out_ref[...] = (jnp.sum(h * W3_ref[...], axis=1, keepdims=True)
                    + b3_ref[0, 0])


def _tail(seq_i32, sums, cat_features, W_cat, b_cat, g_cat, be_cat,
          W1, b1, g1, be1, W2, b2, g2, be2, W3, b3, interpret=False):
    row = lambda x: x.reshape(1, -1)
    return pl.pallas_call(
        _tail_body,
        out_shape=jax.ShapeDtypeStruct((_B, 1), jnp.float32),
        interpret=interpret,
    )(seq_i32, sums, cat_features, W_cat, row(b_cat), row(g_cat),
      row(be_cat), W1, row(b1), row(g1), row(be1), W2, row(b2), row(g2),
      row(be2), W3, row(b3))


def kernel(text_seq, cat_features, emb_table, W_cat, b_cat, g_cat, be_cat,
           W1, b1, g1, be1, W2, b2, g2, be2, W3, b3):
    seq_i32 = text_seq.astype(jnp.int32)
    tbl_lin = _untile(emb_table.T).reshape(2 * _H, _E)
    gidx = jnp.where(seq_i32 < _H, 2 * seq_i32, 2 * seq_i32 - (2 * _H - 1))
    sums = _bag_sums(tbl_lin, gidx.reshape(_B * _S))
    return _tail(seq_i32, sums, cat_features, W_cat, b_cat, g_cat, be_cat,
                 W1, b1, g1, be1, W2, b2, g2, be2, W3, b3)


# speedup vs baseline: 24.9300x; 1.0027x over previous
import functools

import jax
import jax.numpy as jnp
from jax import lax
from jax.experimental import pallas as pl
from jax.experimental.pallas import tpu as pltpu
from jax.experimental.pallas import tpu_sc as plsc

_B, _S, _E = 4096, 200, 64
_NC, _NS = 2, 16
_NW = _NC * _NS
_BPW = _B // _NW
_W0 = 128
_W1 = _S - _W0


_V = 1000000
_H = 524288
_VB = 8192
_NBLK = _H // _VB


def _untile_body(in0_ref, in1_ref, out_ref):
    out_ref[...] = jnp.concatenate([in0_ref[...].T, in1_ref[...].T], axis=1)


def _untile(tbl_t):
    return pl.pallas_call(
        _untile_body,
        grid=(_NBLK,),
        in_specs=[pl.BlockSpec((_E, _VB), lambda i: (0, i)),
                  pl.BlockSpec((_E, _VB),
                               lambda i: (0, jnp.minimum(i + _NBLK,
                                                         _V // _VB - 1)))],
        out_specs=pl.BlockSpec((_VB, 2 * _E), lambda i: (i, 0)),
        out_shape=jax.ShapeDtypeStruct((_H, 2 * _E), jnp.float32),
        compiler_params=pltpu.CompilerParams(
            dimension_semantics=("parallel",)),
    )(tbl_t, tbl_t)


def _bag_sums(emb_table, idx_flat):
    mesh = plsc.VectorSubcoreMesh(
        core_axis_name="c", subcore_axis_name="s",
        num_cores=_NC, num_subcores=_NS)

    @functools.partial(
        pl.kernel,
        out_type=jax.ShapeDtypeStruct((_B, _E), jnp.float32),
        mesh=mesh,
        compiler_params=pltpu.CompilerParams(use_tc_tiling_on_sc=False),
        scratch_types=[
            pltpu.VMEM((_BPW * _S,), jnp.int32),
            pltpu.VMEM((2, _S, _E), jnp.float32),
            pltpu.VMEM((_BPW, _E), jnp.float32),
            pltpu.SemaphoreType.DMA,
            pltpu.SemaphoreType.DMA,
        ],
    )
    def k(table_hbm, idx_hbm, out_hbm, idx_v, rows_v, out_v, sem0, sem1):
        wid = lax.axis_index("s") * _NC + lax.axis_index("c")
        base = wid * _BPW
        pltpu.sync_copy(idx_hbm.at[pl.ds(base * _S, _BPW * _S)], idx_v)

        def descs(r, buf, sem):
            off = r * _S
            return (
                pltpu.make_async_copy(
                    table_hbm.at[idx_v.at[pl.ds(off, _W0)]],
                    rows_v.at[buf].at[pl.ds(0, _W0)], sem),
                pltpu.make_async_copy(
                    table_hbm.at[idx_v.at[pl.ds(off + _W0, _W1)]],
                    rows_v.at[buf].at[pl.ds(_W0, _W1)], sem),
            )

        def issue(r, buf, sem):
            for d in descs(r, buf, sem):
                d.start()

        def wait(r, buf, sem):
            for d in descs(r, buf, sem):
                d.wait()

        def accum(r, buf):
            zero = jnp.zeros((16,), jnp.float32)

            def acc_body(i, carry):
                cs = list(carry)
                for j in range(4):
                    row = i * 4 + j
                    for c in range(4):
                        cs[c] = cs[c] + rows_v[buf, row, pl.ds(c * 16, 16)]
                return tuple(cs)

            acc = lax.fori_loop(0, _S // 4, acc_body, (zero,) * 4)
            for c in range(4):
                out_v[r, pl.ds(c * 16, 16)] = acc[c]

        issue(0, 0, sem0)
        issue(1, 1, sem1)

        @pl.loop(0, _BPW // 2)
        def _pair(i):
            r0 = 2 * i
            wait(r0, 0, sem0)

            @pl.when(i < _BPW // 2 - 1)
            def _():
                issue(r0 + 2, 0, sem0)

            accum(r0, 0)
            r1 = r0 + 1
            wait(r1, 1, sem1)

            @pl.when(i < _BPW // 2 - 1)
            def _():
                issue(r1 + 2, 1, sem1)

            accum(r1, 1)

        pltpu.sync_copy(out_v, out_hbm.at[pl.ds(base, _BPW)])

    return k(emb_table, idx_flat)


def _tail_body(seq_ref, sums_ref, cat_ref, Wc_ref, bc_ref, gc_ref, bec_ref,
               W1_ref, b1_ref, g1_ref, be1_ref, W2_ref, b2_ref, g2_ref,
               be2_ref, W3_ref, b3_ref, out_ref):
    def mm(x, w_ref_val):
        return lax.dot_general(x, w_ref_val, (((1,), (1,)), ((), ())),
                               preferred_element_type=jnp.float32)

    def bn(x, g, b):
        m = jnp.mean(x, axis=0, keepdims=True)
        v = jnp.mean((x - m) ** 2, axis=0, keepdims=True)
        return g * (x - m) * lax.rsqrt(v + 1e-5) + b

    seq = seq_ref[...]
    mask = (seq != 0).astype(jnp.float32)
    cnt = lax.dot_general(mask, jnp.ones((_S, _E), jnp.float32),
                          (((1,), (0,)), ((), ())),
                          preferred_element_type=jnp.float32)
    text_feat = sums_ref[...] / (cnt + 1e-9)

    cat = mm(cat_ref[...], Wc_ref[...]) + bc_ref[...]
    cat = jnp.maximum(bn(cat, gc_ref[...], bec_ref[...]), 0.0)

    h = (mm(text_feat, W1_ref[:, : _E]) + mm(cat, W1_ref[:, _E:])
         + b1_ref[...])
    h = jnp.maximum(bn(h, g1_ref[...], be1_ref[...]), 0.0)
    h = mm(h, W2_ref[...]) + b2_ref[...]
    h = jnp.maximum(bn(h, g2_ref[...], be2_ref[...]), 0.0)
    out_ref[...] = (jnp.sum(h * W3_ref[...], axis=1, keepdims=True)
                    + b3_ref[0, 0])


def _tail(seq_i32, sums, cat_features, W_cat, b_cat, g_cat, be_cat,
          W1, b1, g1, be1, W2, b2, g2, be2, W3, b3, interpret=False):
    row = lambda x: x.reshape(1, -1)
    return pl.pallas_call(
        _tail_body,
        out_shape=jax.ShapeDtypeStruct((_B, 1), jnp.float32),
        interpret=interpret,
    )(seq_i32, sums, cat_features, W_cat, row(b_cat), row(g_cat),
      row(be_cat), W1, row(b1), row(g1), row(be1), W2, row(b2), row(g2),
      row(be2), W3, row(b3))


def kernel(text_seq, cat_features, emb_table, W_cat, b_cat, g_cat, be_cat,
           W1, b1, g1, be1, W2, b2, g2, be2, W3, b3):
    seq_i32 = text_seq.astype(jnp.int32)
    tbl_lin = _untile(emb_table.T).reshape(2 * _H, _E)
    gidx = jnp.where(seq_i32 < _H, 2 * seq_i32, 2 * seq_i32 - (2 * _H - 1))
    sums = _bag_sums(tbl_lin, gidx.reshape(_B * _S))
    return _tail(seq_i32, sums, cat_features, W_cat, b_cat, g_cat, be_cat,
                 W1, b1, g1, be1, W2, b2, g2, be2, W3, b3)


# speedup vs baseline: 25.9414x; 1.0406x over previous
import functools

import jax
import jax.numpy as jnp
from jax import lax
from jax.experimental import pallas as pl
from jax.experimental.pallas import tpu as pltpu
from jax.experimental.pallas import tpu_sc as plsc

_B, _S, _E = 4096, 200, 64
_NC, _NS = 2, 16
_NW = _NC * _NS
_BPW = _B // _NW
_W0 = 128
_W1 = _S - _W0


_V = 1000000
_H = 524288
_VB = 16384
_NBLK = _H // _VB


def _untile_body(in0_ref, in1_ref, out_ref):
    out_ref[...] = jnp.concatenate([in0_ref[...].T, in1_ref[...].T], axis=1)


def _untile(tbl_t):
    return pl.pallas_call(
        _untile_body,
        grid=(_NBLK,),
        in_specs=[pl.BlockSpec((_E, _VB), lambda i: (0, i)),
                  pl.BlockSpec((_E, _VB),
                               lambda i: (0, jnp.minimum(i + _NBLK,
                                                         _V // _VB - 1)))],
        out_specs=pl.BlockSpec((_VB, 2 * _E), lambda i: (i, 0)),
        out_shape=jax.ShapeDtypeStruct((_H, 2 * _E), jnp.float32),
    )(tbl_t, tbl_t)


def _bag_sums(emb_table, idx_flat):
    mesh = plsc.VectorSubcoreMesh(
        core_axis_name="c", subcore_axis_name="s",
        num_cores=_NC, num_subcores=_NS)

    @functools.partial(
        pl.kernel,
        out_type=jax.ShapeDtypeStruct((_B, _E), jnp.float32),
        mesh=mesh,
        compiler_params=pltpu.CompilerParams(use_tc_tiling_on_sc=False),
        scratch_types=[
            pltpu.VMEM((_BPW * _S,), jnp.int32),
            pltpu.VMEM((2, _S, _E), jnp.float32),
            pltpu.VMEM((_BPW, _E), jnp.float32),
            pltpu.SemaphoreType.DMA,
            pltpu.SemaphoreType.DMA,
        ],
    )
    def k(table_hbm, idx_hbm, out_hbm, idx_v, rows_v, out_v, sem0, sem1):
        wid = lax.axis_index("s") * _NC + lax.axis_index("c")
        base = wid * _BPW
        pltpu.sync_copy(idx_hbm.at[pl.ds(base * _S, _BPW * _S)], idx_v)

        def descs(r, buf, sem):
            off = r * _S
            return (
                pltpu.make_async_copy(
                    table_hbm.at[idx_v.at[pl.ds(off, _W0)]],
                    rows_v.at[buf].at[pl.ds(0, _W0)], sem),
                pltpu.make_async_copy(
                    table_hbm.at[idx_v.at[pl.ds(off + _W0, _W1)]],
                    rows_v.at[buf].at[pl.ds(_W0, _W1)], sem),
            )

        def issue(r, buf, sem):
            for d in descs(r, buf, sem):
                d.start()

        def wait(r, buf, sem):
            for d in descs(r, buf, sem):
                d.wait()

        def accum(r, buf):
            zero = jnp.zeros((16,), jnp.float32)

            def acc_body(i, carry):
                cs = list(carry)
                for j in range(4):
                    row = i * 4 + j
                    for c in range(4):
                        cs[c] = cs[c] + rows_v[buf, row, pl.ds(c * 16, 16)]
                return tuple(cs)

            acc = lax.fori_loop(0, _S // 4, acc_body, (zero,) * 4)
            for c in range(4):
                out_v[r, pl.ds(c * 16, 16)] = acc[c]

        issue(0, 0, sem0)
        issue(1, 1, sem1)

        @pl.loop(0, _BPW // 2)
        def _pair(i):
            r0 = 2 * i
            wait(r0, 0, sem0)

            @pl.when(i < _BPW // 2 - 1)
            def _():
                issue(r0 + 2, 0, sem0)

            accum(r0, 0)
            r1 = r0 + 1
            wait(r1, 1, sem1)

            @pl.when(i < _BPW // 2 - 1)
            def _():
                issue(r1 + 2, 1, sem1)

            accum(r1, 1)

        pltpu.sync_copy(out_v, out_hbm.at[pl.ds(base, _BPW)])

    return k(emb_table, idx_flat)


def _tail_body(seq_ref, sums_ref, cat_ref, Wc_ref, bc_ref, gc_ref, bec_ref,
               W1_ref, b1_ref, g1_ref, be1_ref, W2_ref, b2_ref, g2_ref,
               be2_ref, W3_ref, b3_ref, out_ref):
    def mm(x, w_ref_val):
        return lax.dot_general(x, w_ref_val, (((1,), (1,)), ((), ())),
                               preferred_element_type=jnp.float32)

    def bn(x, g, b):
        m = jnp.mean(x, axis=0, keepdims=True)
        v = jnp.mean((x - m) ** 2, axis=0, keepdims=True)
        return g * (x - m) * lax.rsqrt(v + 1e-5) + b

    seq = seq_ref[...]
    mask = (seq != 0).astype(jnp.float32)
    cnt = lax.dot_general(mask, jnp.ones((_S, _E), jnp.float32),
                          (((1,), (0,)), ((), ())),
                          preferred_element_type=jnp.float32)
    text_feat = sums_ref[...] / (cnt + 1e-9)

    cat = mm(cat_ref[...], Wc_ref[...]) + bc_ref[...]
    cat = jnp.maximum(bn(cat, gc_ref[...], bec_ref[...]), 0.0)

    h = (mm(text_feat, W1_ref[:, : _E]) + mm(cat, W1_ref[:, _E:])
         + b1_ref[...])
    h = jnp.maximum(bn(h, g1_ref[...], be1_ref[...]), 0.0)
    h = mm(h, W2_ref[...]) + b2_ref[...]
    h = jnp.maximum(bn(h, g2_ref[...], be2_ref[...]), 0.0)
    out_ref[...] = (jnp.sum(h * W3_ref[...], axis=1, keepdims=True)
                    + b3_ref[0, 0])


def _tail(seq_i32, sums, cat_features, W_cat, b_cat, g_cat, be_cat,
          W1, b1, g1, be1, W2, b2, g2, be2, W3, b3, interpret=False):
    row = lambda x: x.reshape(1, -1)
    return pl.pallas_call(
        _tail_body,
        out_shape=jax.ShapeDtypeStruct((_B, 1), jnp.float32),
        interpret=interpret,
    )(seq_i32, sums, cat_features, W_cat, row(b_cat), row(g_cat),
      row(be_cat), W1, row(b1), row(g1), row(be1), W2, row(b2), row(g2),
      row(be2), W3, row(b3))


def kernel(text_seq, cat_features, emb_table, W_cat, b_cat, g_cat, be_cat,
           W1, b1, g1, be1, W2, b2, g2, be2, W3, b3):
    seq_i32 = text_seq.astype(jnp.int32)
    tbl_lin = _untile(emb_table.T).reshape(2 * _H, _E)
    gidx = jnp.where(seq_i32 < _H, 2 * seq_i32, 2 * seq_i32 - (2 * _H - 1))
    sums = _bag_sums(tbl_lin, gidx.reshape(_B * _S))
    return _tail(seq_i32, sums, cat_features, W_cat, b_cat, g_cat, be_cat,
                 W1, b1, g1, be1, W2, b2, g2, be2, W3, b3)
